# Initial kernel scaffold; baseline (speedup 1.0000x reference)
import jax
import jax.numpy as jnp
from jax.experimental import pallas as pl


def kernel(hidden_states, transformer_output, W_gat, a_src, a_dst, b_gat, W_fus, b_fus):
    raise NotImplementedError("write your pallas kernel here")


# speedup vs baseline: 176.5068x; 176.5068x over previous
import functools

import jax
import jax.numpy as jnp
from jax.experimental import pallas as pl

B, S, H = 4, 128, 768
NEG_SLOPE = 0.2


def _gat_kernel(x_ref, t_ref, wg_ref, a2_ref, bg_ref, wf_ref, bf_ref, out_ref):
    x = x_ref[0]
    t = t_ref[0]

    xp = jnp.dot(x, wg_ref[...], preferred_element_type=jnp.float32)
    sa = jnp.dot(xp, a2_ref[...], preferred_element_type=jnp.float32)
    alpha_s = sa[:, 0]
    alpha_d = sa[:, 1]

    logits = alpha_d[:, None] + alpha_s[None, :]
    logits = jnp.where(logits > 0, logits, NEG_SLOPE * logits)
    row = jax.lax.broadcasted_iota(jnp.int32, (S, S), 0)
    col = jax.lax.broadcasted_iota(jnp.int32, (S, S), 1)
    logits = jnp.where(row == col, -jnp.inf, logits)

    m = jnp.max(logits, axis=1, keepdims=True)
    ex = jnp.exp(logits - m)
    attn = ex / jnp.sum(ex, axis=1, keepdims=True)

    g = jnp.dot(attn, xp, preferred_element_type=jnp.float32) + bg_ref[...]
    fused = jnp.concatenate([t, g], axis=-1)
    out = jnp.dot(fused, wf_ref[...], preferred_element_type=jnp.float32)
    out_ref[0] = out + bf_ref[...]


@functools.partial(jax.jit, static_argnames=())
def kernel(hidden_states, transformer_output, W_gat, a_src, a_dst, b_gat, W_fus, b_fus):
    a2 = jnp.stack([a_src, a_dst], axis=1)
    bg = b_gat.reshape(1, H)
    bf = b_fus.reshape(1, H)

    return pl.pallas_call(
        _gat_kernel,
        grid=(B,),
        in_specs=[
            pl.BlockSpec((1, S, H), lambda b: (b, 0, 0)),
            pl.BlockSpec((1, S, H), lambda b: (b, 0, 0)),
            pl.BlockSpec((H, H), lambda b: (0, 0)),
            pl.BlockSpec((H, 2), lambda b: (0, 0)),
            pl.BlockSpec((1, H), lambda b: (0, 0)),
            pl.BlockSpec((2 * H, H), lambda b: (0, 0)),
            pl.BlockSpec((1, H), lambda b: (0, 0)),
        ],
        out_specs=pl.BlockSpec((1, S, H), lambda b: (b, 0, 0)),
        out_shape=jax.ShapeDtypeStruct((B, S, H), jnp.float32),
    )(hidden_states, transformer_output, W_gat, a2, bg, W_fus, bf)
